# Initial kernel scaffold; baseline (speedup 1.0000x reference)
import jax
import jax.numpy as jnp
from jax.experimental import pallas as pl


def kernel(f, target_number_points, nodes):
    raise NotImplementedError("write your pallas kernel here")


# speedup vs baseline: 307.0851x; 307.0851x over previous
import functools

import jax
import jax.numpy as jnp
from jax import lax
from jax.experimental import pallas as pl
from jax.experimental.pallas import tpu as pltpu
from jax.experimental.pallas import tpu_sc as plsc

N = 100000
NPAD = 100352
SEG = NPAD // 16
D = 128
KSEL = 16384
TOTAL = 10 * N
UPAD = 1003520
WU = UPAD // 32
UCH = 6272
NCH = WU // UCH
NBINS = 2048
SEARCH_STEPS = 17


def _iota16():
    return lax.broadcasted_iota(jnp.int32, (16,), 0)


def _dup_rank(ci, tmp16):
    iota = _iota16()
    tmp16[...] = ci
    r = jnp.zeros((16,), jnp.int32)
    tot = jnp.zeros((16,), jnp.int32)
    one = jnp.ones((16,), jnp.int32)
    zero = jnp.zeros((16,), jnp.int32)
    for sft in range(1, 16):
        perm = (iota - sft) & 15
        cs = plsc.load_gather(tmp16, [perm])
        e = cs == ci
        r = r + jnp.where(e & (iota >= sft), one, zero)
        tot = tot + jnp.where(e, one, zero)
    last = tot == r
    return r, last


def _k1_body(cdf_hbm, u_hbm, zseg_hbm, ones_hbm, counts_hbm,
             cdf_v, u_v, draws_v, ones_v, counts_sh, sem):
    c = lax.axis_index("c")
    s = lax.axis_index("s")
    wid = c * 16 + s
    pltpu.sync_copy(cdf_hbm, cdf_v)
    pltpu.sync_copy(ones_hbm, ones_v)
    pltpu.sync_copy(zseg_hbm, counts_sh.at[pl.ds(s * SEG, SEG)])
    plsc.subcore_barrier()

    for ch in range(NCH):
        ubase = wid * WU + ch * UCH
        pltpu.sync_copy(u_hbm.at[pl.ds(ubase, UCH)], u_v)

        def search_row(j, _):
            for t in range(8):
                off = j * 128 + t * 16
                uvec = u_v[pl.ds(off, 16)]
                lo = jnp.zeros((16,), jnp.int32)
                hi = jnp.full((16,), N, jnp.int32)
                for _step in range(SEARCH_STEPS):
                    mid = (lo + hi) >> 1
                    cv = plsc.load_gather(cdf_v, [mid])
                    take = cv < uvec
                    lo = jnp.where(take, mid + 1, lo)
                    hi = jnp.where(take, hi, mid)
                draws_v[j, 16 * t:16 * (t + 1)] = lo
            return 0

        lax.fori_loop(0, UCH // 128, search_row, 0)

        def scat_row(j, _):
            pltpu.sync_copy(ones_v, counts_sh.at[draws_v.at[j]], add=True)
            return 0

        lax.fori_loop(0, UCH // 128, scat_row, 0)

    plsc.subcore_barrier()
    pltpu.sync_copy(counts_sh.at[pl.ds(s * SEG, SEG)],
                    counts_hbm.at[c, pl.ds(s * SEG, SEG)])


def _make_k1():
    return functools.partial(
        pl.kernel,
        out_type=jax.ShapeDtypeStruct((2, NPAD), jnp.float32),
        mesh=plsc.VectorSubcoreMesh(core_axis_name="c", subcore_axis_name="s",
                                    num_cores=2, num_subcores=16),
        compiler_params=pltpu.CompilerParams(needs_layout_passes=False),
        scratch_types=[
            pltpu.VMEM((NPAD,), jnp.float32),
            pltpu.VMEM((UCH,), jnp.float32),
            pltpu.VMEM((UCH // 128, 128), jnp.int32),
            pltpu.VMEM((128,), jnp.float32),
            pltpu.VMEM_SHARED((NPAD,), jnp.float32),
            pltpu.SemaphoreType.DMA,
        ],
    )(_k1_body)


def _k2_body(counts_hbm, idx_hbm,
             ca_v, cb_v, cls_v, hist_v, habuf_v, gsum_v, offb_v,
             posb_v, catb_v, tmp16_v, hista_sh, outsort_sh, sem):
    w = lax.axis_index("s")
    base = w * SEG
    iota = _iota16()
    pltpu.sync_copy(counts_hbm.at[0, pl.ds(base, SEG)], ca_v)
    pltpu.sync_copy(counts_hbm.at[1, pl.ds(base, SEG)], cb_v)

    def zero_hist(k, _):
        hist_v[pl.ds(k * 16, 16)] = jnp.zeros((16,), jnp.int32)
        return 0

    lax.fori_loop(0, NBINS // 16, zero_hist, 0)

    def phase_a(j, _):
        for t in range(8):
            off = j * 128 + t * 16
            cnt = ca_v[pl.ds(off, 16)] + cb_v[pl.ds(off, 16)]
            ci = jnp.minimum(cnt.astype(jnp.int32),
                             jnp.full((16,), NBINS - 1, jnp.int32))
            gidx = base + off + iota
            ci = jnp.where(gidx < N, ci, jnp.zeros((16,), jnp.int32))
            cls_v[pl.ds(off, 16)] = ci
            r, last = _dup_rank(ci, tmp16_v)
            plsc.addupdate_scatter(hist_v, [ci], r + 1, mask=last)
        return 0

    lax.fori_loop(0, SEG // 128, phase_a, 0)

    pltpu.sync_copy(hist_v, hista_sh.at[pl.ds(w * NBINS, NBINS)])
    plsc.subcore_barrier()
    pltpu.sync_copy(hista_sh, habuf_v)

    def merge(k, _):
        g = jnp.zeros((16,), jnp.int32)
        my = jnp.zeros((16,), jnp.int32)
        for t in range(16):
            h = habuf_v[pl.ds(t * NBINS + k * 16, 16)]
            g = g + h
            my = my + jnp.where(w > t, h, jnp.zeros((16,), jnp.int32))
        gsum_v[pl.ds(k * 16, 16)] = g
        offb_v[pl.ds(k * 16, 16)] = my
        return 0

    lax.fori_loop(0, NBINS // 16, merge, 0)

    def suffix(k, carry):
        g = gsum_v[pl.ds(k * 16, 16)]
        incl = plsc.cumsum(g) + carry
        start = jnp.full((16,), NPAD, jnp.int32) - incl
        offb_v[pl.ds(k * 16, 16)] = offb_v[pl.ds(k * 16, 16)] + start
        return carry + jnp.sum(g)

    lax.fori_loop(0, NBINS // 16, suffix, jnp.int32(0))

    def phase_c(j, _):
        for t in range(8):
            off = j * 128 + t * 16
            ci = cls_v[pl.ds(off, 16)]
            r, last = _dup_rank(ci, tmp16_v)
            g = plsc.load_gather(offb_v, [ci])
            pos = g + r
            cat = base + off + iota
            posb_v[j, 16 * t:16 * (t + 1)] = pos
            catb_v[j, 16 * t:16 * (t + 1)] = cat
            plsc.store_scatter(offb_v, [ci], pos + 1, mask=last)
        return 0

    lax.fori_loop(0, SEG // 128, phase_c, 0)

    def scat_row(j, _):
        pltpu.sync_copy(catb_v.at[j], outsort_sh.at[posb_v.at[j]])
        return 0

    lax.fori_loop(0, SEG // 128, scat_row, 0)
    plsc.subcore_barrier()

    @pl.when(w < 4)
    def _():
        pltpu.sync_copy(outsort_sh.at[pl.ds(w * 4096, 4096)],
                        idx_hbm.at[pl.ds(w * 4096, 4096)])


def _make_k2():
    return functools.partial(
        pl.kernel,
        out_type=jax.ShapeDtypeStruct((KSEL,), jnp.int32),
        mesh=plsc.VectorSubcoreMesh(core_axis_name="c", subcore_axis_name="s",
                                    num_cores=1, num_subcores=16),
        compiler_params=pltpu.CompilerParams(needs_layout_passes=False),
        scratch_types=[
            pltpu.VMEM((SEG,), jnp.float32),
            pltpu.VMEM((SEG,), jnp.float32),
            pltpu.VMEM((SEG,), jnp.int32),
            pltpu.VMEM((NBINS,), jnp.int32),
            pltpu.VMEM((16 * NBINS,), jnp.int32),
            pltpu.VMEM((NBINS,), jnp.int32),
            pltpu.VMEM((NBINS,), jnp.int32),
            pltpu.VMEM((SEG // 128, 128), jnp.int32),
            pltpu.VMEM((SEG // 128, 128), jnp.int32),
            pltpu.VMEM((16,), jnp.int32),
            pltpu.VMEM_SHARED((16 * NBINS,), jnp.int32),
            pltpu.VMEM_SHARED((NPAD,), jnp.int32),
            pltpu.SemaphoreType.DMA,
        ],
    )(_k2_body)


def _k3_body(idx_hbm, nodes_hbm, out_hbm, idx_v, rows_v, sem):
    c = lax.axis_index("c")
    s = lax.axis_index("s")
    wid = c * 16 + s
    pltpu.sync_copy(idx_hbm.at[pl.ds(wid * 4, 4)], idx_v)
    for j in range(4):
        pltpu.async_copy(nodes_hbm.at[idx_v.at[j]],
                         rows_v.at[pl.ds(j * 128, 128)], sem).wait()
    pltpu.sync_copy(rows_v, out_hbm.at[pl.ds(wid * 512, 512)])


def _make_k3():
    return functools.partial(
        pl.kernel,
        out_type=jax.ShapeDtypeStruct((KSEL, D), jnp.float32),
        mesh=plsc.VectorSubcoreMesh(core_axis_name="c", subcore_axis_name="s",
                                    num_cores=2, num_subcores=16),
        compiler_params=pltpu.CompilerParams(needs_layout_passes=False),
        scratch_types=[
            pltpu.VMEM((4, 128), jnp.int32),
            pltpu.VMEM((512, D), jnp.float32),
            pltpu.SemaphoreType.DMA,
        ],
    )(_k3_body)


@functools.cache
def _kernels():
    return _make_k1(), _make_k2(), _make_k3()


def kernel(f, target_number_points, nodes):
    probs = f / jnp.sum(f)
    cdf = jnp.cumsum(probs)
    u = jax.random.uniform(jax.random.key(42), (TOTAL,),
                           dtype=jnp.float32) * cdf[-1]
    u_pad = jnp.concatenate(
        [u, jnp.full((UPAD - TOTAL,), jnp.inf, jnp.float32)])
    cdf_pad = jnp.concatenate([cdf, jnp.zeros((NPAD - N,), jnp.float32)])
    zeros_seg = jnp.zeros((SEG,), jnp.float32)
    ones128 = jnp.ones((128,), jnp.float32)
    k1, k2, k3 = _kernels()
    counts2 = k1(cdf_pad, u_pad, zeros_seg, ones128)
    sel = k2(counts2)
    sel = (sel + (target_number_points - KSEL)).astype(jnp.int32)
    return k3(sel.reshape(128, 128), nodes)


# speedup vs baseline: 320.1236x; 1.0425x over previous
import functools

import jax
import jax.numpy as jnp
from jax import lax
from jax.experimental import pallas as pl
from jax.experimental.pallas import tpu as pltpu
from jax.experimental.pallas import tpu_sc as plsc

N = 100000
NPAD = 100352
SEG = NPAD // 16
D = 128
KSEL = 16384
TOTAL = 10 * N
UPAD = 1003520
WU = UPAD // 32
UCH = 6272
NCH = WU // UCH
NBINS = 2048
SEARCH_STEPS = 17


def _iota16():
    return lax.broadcasted_iota(jnp.int32, (16,), 0)


def _dup_rank(ci, tmp16):
    iota = _iota16()
    tmp16[...] = ci
    r = jnp.zeros((16,), jnp.int32)
    tot = jnp.zeros((16,), jnp.int32)
    one = jnp.ones((16,), jnp.int32)
    zero = jnp.zeros((16,), jnp.int32)
    for sft in range(1, 16):
        perm = (iota - sft) & 15
        cs = plsc.load_gather(tmp16, [perm])
        e = cs == ci
        r = r + jnp.where(e & (iota >= sft), one, zero)
        tot = tot + jnp.where(e, one, zero)
    last = tot == r
    return r, last


def _k1_body(cdf_hbm, u_hbm, zseg_hbm, ones_hbm, counts_hbm,
             cdf_v, u_v, draws_a, draws_b, ones_v, counts_sh, sem):
    c = lax.axis_index("c")
    s = lax.axis_index("s")
    wid = c * 16 + s
    pltpu.sync_copy(cdf_hbm, cdf_v)
    pltpu.sync_copy(ones_hbm, ones_v)
    pltpu.sync_copy(zseg_hbm, counts_sh.at[pl.ds(s * SEG, SEG)])
    plsc.subcore_barrier()

    pending = [[], []]
    for ch in range(NCH):
        pb = ch % 2
        ubase = wid * WU + ch * UCH
        pltpu.sync_copy(u_hbm.at[pl.ds(ubase, UCH)], u_v)
        for h in pending[pb]:
            h.wait()
        pending[pb] = []

        draws_v = (draws_a, draws_b)[pb]

        def search_row(j, _, draws_v=draws_v):
            for t in range(8):
                off = j * 128 + t * 16
                uvec = u_v[pl.ds(off, 16)]
                lo = jnp.zeros((16,), jnp.int32)
                hi = jnp.full((16,), N, jnp.int32)
                for _step in range(SEARCH_STEPS):
                    mid = (lo + hi) >> 1
                    cv = plsc.load_gather(cdf_v, [mid])
                    take = cv < uvec
                    lo = jnp.where(take, mid + 1, lo)
                    hi = jnp.where(take, hi, mid)
                draws_v[j, 16 * t:16 * (t + 1)] = lo
            return 0

        lax.fori_loop(0, UCH // 128, search_row, 0)

        pending[pb] = [
            pltpu.async_copy(ones_v, counts_sh.at[draws_v.at[j]],
                             sem, add=True)
            for j in range(UCH // 128)
        ]

    for hs in pending:
        for h in hs:
            h.wait()
    plsc.subcore_barrier()
    pltpu.sync_copy(counts_sh.at[pl.ds(s * SEG, SEG)],
                    counts_hbm.at[c, pl.ds(s * SEG, SEG)])


def _make_k1():
    return functools.partial(
        pl.kernel,
        out_type=jax.ShapeDtypeStruct((2, NPAD), jnp.float32),
        mesh=plsc.VectorSubcoreMesh(core_axis_name="c", subcore_axis_name="s",
                                    num_cores=2, num_subcores=16),
        compiler_params=pltpu.CompilerParams(needs_layout_passes=False),
        scratch_types=[
            pltpu.VMEM((NPAD,), jnp.float32),
            pltpu.VMEM((UCH,), jnp.float32),
            pltpu.VMEM((UCH // 128, 128), jnp.int32),
            pltpu.VMEM((UCH // 128, 128), jnp.int32),
            pltpu.VMEM((128,), jnp.float32),
            pltpu.VMEM_SHARED((NPAD,), jnp.float32),
            pltpu.SemaphoreType.DMA,
        ],
    )(_k1_body)


def _k2_body(counts_hbm, idx_hbm,
             ca_v, cb_v, cls_v, hist_v, habuf_v, gsum_v, offb_v,
             posb_v, catb_v, tmp16_v, hista_sh, outsort_sh, sem):
    w = lax.axis_index("s")
    base = w * SEG
    iota = _iota16()
    pltpu.sync_copy(counts_hbm.at[0, pl.ds(base, SEG)], ca_v)
    pltpu.sync_copy(counts_hbm.at[1, pl.ds(base, SEG)], cb_v)

    def zero_hist(k, _):
        hist_v[pl.ds(k * 16, 16)] = jnp.zeros((16,), jnp.int32)
        return 0

    lax.fori_loop(0, NBINS // 16, zero_hist, 0)

    def phase_a(j, _):
        for t in range(8):
            off = j * 128 + t * 16
            cnt = ca_v[pl.ds(off, 16)] + cb_v[pl.ds(off, 16)]
            ci = jnp.minimum(cnt.astype(jnp.int32),
                             jnp.full((16,), NBINS - 1, jnp.int32))
            gidx = base + off + iota
            ci = jnp.where(gidx < N, ci, jnp.zeros((16,), jnp.int32))
            cls_v[pl.ds(off, 16)] = ci
            r, last = _dup_rank(ci, tmp16_v)
            plsc.addupdate_scatter(hist_v, [ci], r + 1, mask=last)
        return 0

    lax.fori_loop(0, SEG // 128, phase_a, 0)

    pltpu.sync_copy(hist_v, hista_sh.at[pl.ds(w * NBINS, NBINS)])
    plsc.subcore_barrier()
    pltpu.sync_copy(hista_sh, habuf_v)

    def merge(k, _):
        g = jnp.zeros((16,), jnp.int32)
        my = jnp.zeros((16,), jnp.int32)
        for t in range(16):
            h = habuf_v[pl.ds(t * NBINS + k * 16, 16)]
            g = g + h
            my = my + jnp.where(w > t, h, jnp.zeros((16,), jnp.int32))
        gsum_v[pl.ds(k * 16, 16)] = g
        offb_v[pl.ds(k * 16, 16)] = my
        return 0

    lax.fori_loop(0, NBINS // 16, merge, 0)

    def suffix(k, carry):
        g = gsum_v[pl.ds(k * 16, 16)]
        incl = plsc.cumsum(g) + carry
        start = jnp.full((16,), NPAD, jnp.int32) - incl
        offb_v[pl.ds(k * 16, 16)] = offb_v[pl.ds(k * 16, 16)] + start
        return carry + jnp.sum(g)

    lax.fori_loop(0, NBINS // 16, suffix, jnp.int32(0))

    def phase_c(j, _):
        for t in range(8):
            off = j * 128 + t * 16
            ci = cls_v[pl.ds(off, 16)]
            r, last = _dup_rank(ci, tmp16_v)
            g = plsc.load_gather(offb_v, [ci])
            pos = g + r
            cat = base + off + iota
            posb_v[j, 16 * t:16 * (t + 1)] = pos
            catb_v[j, 16 * t:16 * (t + 1)] = cat
            plsc.store_scatter(offb_v, [ci], pos + 1, mask=last)
        return 0

    lax.fori_loop(0, SEG // 128, phase_c, 0)

    def scat_row(j, _):
        pltpu.sync_copy(catb_v.at[j], outsort_sh.at[posb_v.at[j]])
        return 0

    lax.fori_loop(0, SEG // 128, scat_row, 0)
    plsc.subcore_barrier()

    @pl.when(w < 4)
    def _():
        pltpu.sync_copy(outsort_sh.at[pl.ds(w * 4096, 4096)],
                        idx_hbm.at[pl.ds(w * 4096, 4096)])


def _make_k2():
    return functools.partial(
        pl.kernel,
        out_type=jax.ShapeDtypeStruct((KSEL,), jnp.int32),
        mesh=plsc.VectorSubcoreMesh(core_axis_name="c", subcore_axis_name="s",
                                    num_cores=1, num_subcores=16),
        compiler_params=pltpu.CompilerParams(needs_layout_passes=False),
        scratch_types=[
            pltpu.VMEM((SEG,), jnp.float32),
            pltpu.VMEM((SEG,), jnp.float32),
            pltpu.VMEM((SEG,), jnp.int32),
            pltpu.VMEM((NBINS,), jnp.int32),
            pltpu.VMEM((16 * NBINS,), jnp.int32),
            pltpu.VMEM((NBINS,), jnp.int32),
            pltpu.VMEM((NBINS,), jnp.int32),
            pltpu.VMEM((SEG // 128, 128), jnp.int32),
            pltpu.VMEM((SEG // 128, 128), jnp.int32),
            pltpu.VMEM((16,), jnp.int32),
            pltpu.VMEM_SHARED((16 * NBINS,), jnp.int32),
            pltpu.VMEM_SHARED((NPAD,), jnp.int32),
            pltpu.SemaphoreType.DMA,
        ],
    )(_k2_body)


def _k3_body(idx_hbm, nodes_hbm, out_hbm, idx_v, rows_v, sem):
    c = lax.axis_index("c")
    s = lax.axis_index("s")
    wid = c * 16 + s
    pltpu.sync_copy(idx_hbm.at[pl.ds(wid * 4, 4)], idx_v)
    for j in range(4):
        pltpu.async_copy(nodes_hbm.at[idx_v.at[j]],
                         rows_v.at[pl.ds(j * 128, 128)], sem).wait()
    pltpu.sync_copy(rows_v, out_hbm.at[pl.ds(wid * 512, 512)])


def _make_k3():
    return functools.partial(
        pl.kernel,
        out_type=jax.ShapeDtypeStruct((KSEL, D), jnp.float32),
        mesh=plsc.VectorSubcoreMesh(core_axis_name="c", subcore_axis_name="s",
                                    num_cores=2, num_subcores=16),
        compiler_params=pltpu.CompilerParams(needs_layout_passes=False),
        scratch_types=[
            pltpu.VMEM((4, 128), jnp.int32),
            pltpu.VMEM((512, D), jnp.float32),
            pltpu.SemaphoreType.DMA,
        ],
    )(_k3_body)


@functools.cache
def _kernels():
    return _make_k1(), _make_k2(), _make_k3()


def kernel(f, target_number_points, nodes):
    probs = f / jnp.sum(f)
    cdf = jnp.cumsum(probs)
    u = jax.random.uniform(jax.random.key(42), (TOTAL,),
                           dtype=jnp.float32) * cdf[-1]
    u_pad = jnp.concatenate(
        [u, jnp.full((UPAD - TOTAL,), jnp.inf, jnp.float32)])
    cdf_pad = jnp.concatenate([cdf, jnp.zeros((NPAD - N,), jnp.float32)])
    zeros_seg = jnp.zeros((SEG,), jnp.float32)
    ones128 = jnp.ones((128,), jnp.float32)
    k1, k2, k3 = _kernels()
    counts2 = k1(cdf_pad, u_pad, zeros_seg, ones128)
    sel = k2(counts2)
    sel = (sel + (target_number_points - KSEL)).astype(jnp.int32)
    return k3(sel.reshape(128, 128), nodes)


# speedup vs baseline: 715.0801x; 2.2338x over previous
import functools

import jax
import jax.numpy as jnp
from jax import lax
from jax.experimental import pallas as pl
from jax.experimental.pallas import tpu as pltpu
from jax.experimental.pallas import tpu_sc as plsc

N = 100000
NPAD = 100352
SEG = NPAD // 16
D = 128
KSEL = 16384
TOTAL = 10 * N
UPAD = 1003520
WU = UPAD // 32
UCH = 6272
NCH = WU // UCH
NBINS = 2048
SEARCH_STEPS = 17


def _iota16():
    return lax.broadcasted_iota(jnp.int32, (16,), 0)


def _dup_rank(ci, tmp16):
    iota = _iota16()
    tmp16[...] = ci
    r = jnp.zeros((16,), jnp.int32)
    tot = jnp.zeros((16,), jnp.int32)
    one = jnp.ones((16,), jnp.int32)
    zero = jnp.zeros((16,), jnp.int32)
    for sft in range(1, 16):
        perm = (iota - sft) & 15
        cs = plsc.load_gather(tmp16, [perm])
        e = cs == ci
        r = r + jnp.where(e & (iota >= sft), one, zero)
        tot = tot + jnp.where(e, one, zero)
    last = tot == r
    return r, last


def _k1_body(cdf_hbm, u_hbm, zseg_hbm, ones_hbm, counts_hbm,
             cdf_v, u_v, draws_a, draws_b, ones_v, counts_sh, sem):
    c = lax.axis_index("c")
    s = lax.axis_index("s")
    wid = c * 16 + s
    pltpu.sync_copy(cdf_hbm, cdf_v)
    pltpu.sync_copy(ones_hbm, ones_v)
    pltpu.sync_copy(zseg_hbm, counts_sh.at[pl.ds(s * SEG, SEG)])
    plsc.subcore_barrier()

    pending = [[], []]
    for ch in range(NCH):
        pb = ch % 2
        ubase = wid * WU + ch * UCH
        pltpu.sync_copy(u_hbm.at[pl.ds(ubase, UCH)], u_v)
        for h in pending[pb]:
            h.wait()
        pending[pb] = []

        draws_v = (draws_a, draws_b)[pb]

        def search_row(j, _, draws_v=draws_v):
            uvecs = [u_v[pl.ds(j * 128 + t * 16, 16)] for t in range(8)]
            los = [jnp.zeros((16,), jnp.int32) for _ in range(8)]
            his = [jnp.full((16,), N, jnp.int32) for _ in range(8)]
            for _step in range(SEARCH_STEPS):
                for t in range(8):
                    mid = (los[t] + his[t]) >> 1
                    cv = plsc.load_gather(cdf_v, [mid])
                    take = cv < uvecs[t]
                    los[t] = jnp.where(take, mid + 1, los[t])
                    his[t] = jnp.where(take, his[t], mid)
            for t in range(8):
                draws_v[j, 16 * t:16 * (t + 1)] = los[t]
            return 0

        lax.fori_loop(0, UCH // 128, search_row, 0)

        pending[pb] = [
            pltpu.async_copy(ones_v, counts_sh.at[draws_v.at[j]],
                             sem, add=True)
            for j in range(UCH // 128)
        ]

    for hs in pending:
        for h in hs:
            h.wait()
    plsc.subcore_barrier()
    pltpu.sync_copy(counts_sh.at[pl.ds(s * SEG, SEG)],
                    counts_hbm.at[c, pl.ds(s * SEG, SEG)])


def _make_k1():
    return functools.partial(
        pl.kernel,
        out_type=jax.ShapeDtypeStruct((2, NPAD), jnp.float32),
        mesh=plsc.VectorSubcoreMesh(core_axis_name="c", subcore_axis_name="s",
                                    num_cores=2, num_subcores=16),
        compiler_params=pltpu.CompilerParams(needs_layout_passes=False),
        scratch_types=[
            pltpu.VMEM((NPAD,), jnp.float32),
            pltpu.VMEM((UCH,), jnp.float32),
            pltpu.VMEM((UCH // 128, 128), jnp.int32),
            pltpu.VMEM((UCH // 128, 128), jnp.int32),
            pltpu.VMEM((128,), jnp.float32),
            pltpu.VMEM_SHARED((NPAD,), jnp.float32),
            pltpu.SemaphoreType.DMA,
        ],
    )(_k1_body)


def _k2_body(counts_hbm, idx_hbm,
             ca_v, cb_v, cls_v, hist_v, habuf_v, gsum_v, offb_v,
             posb_v, catb_v, tmp16_v, hista_sh, outsort_sh, sem):
    w = lax.axis_index("s")
    base = w * SEG
    iota = _iota16()
    pltpu.sync_copy(counts_hbm.at[0, pl.ds(base, SEG)], ca_v)
    pltpu.sync_copy(counts_hbm.at[1, pl.ds(base, SEG)], cb_v)

    def zero_hist(k, _):
        hist_v[pl.ds(k * 16, 16)] = jnp.zeros((16,), jnp.int32)
        return 0

    lax.fori_loop(0, NBINS // 16, zero_hist, 0)

    def phase_a(j, _):
        for t in range(8):
            off = j * 128 + t * 16
            cnt = ca_v[pl.ds(off, 16)] + cb_v[pl.ds(off, 16)]
            ci = jnp.minimum(cnt.astype(jnp.int32),
                             jnp.full((16,), NBINS - 1, jnp.int32))
            gidx = base + off + iota
            ci = jnp.where(gidx < N, ci, jnp.zeros((16,), jnp.int32))
            cls_v[pl.ds(off, 16)] = ci
            r, last = _dup_rank(ci, tmp16_v)
            plsc.addupdate_scatter(hist_v, [ci], r + 1, mask=last)
        return 0

    lax.fori_loop(0, SEG // 128, phase_a, 0)

    pltpu.sync_copy(hist_v, hista_sh.at[pl.ds(w * NBINS, NBINS)])
    plsc.subcore_barrier()
    pltpu.sync_copy(hista_sh, habuf_v)

    def merge(k, _):
        g = jnp.zeros((16,), jnp.int32)
        my = jnp.zeros((16,), jnp.int32)
        for t in range(16):
            h = habuf_v[pl.ds(t * NBINS + k * 16, 16)]
            g = g + h
            my = my + jnp.where(w > t, h, jnp.zeros((16,), jnp.int32))
        gsum_v[pl.ds(k * 16, 16)] = g
        offb_v[pl.ds(k * 16, 16)] = my
        return 0

    lax.fori_loop(0, NBINS // 16, merge, 0)

    def suffix(k, carry):
        g = gsum_v[pl.ds(k * 16, 16)]
        incl = plsc.cumsum(g) + carry
        start = jnp.full((16,), NPAD, jnp.int32) - incl
        offb_v[pl.ds(k * 16, 16)] = offb_v[pl.ds(k * 16, 16)] + start
        return carry + jnp.sum(g)

    lax.fori_loop(0, NBINS // 16, suffix, jnp.int32(0))

    def phase_c(j, _):
        for t in range(8):
            off = j * 128 + t * 16
            ci = cls_v[pl.ds(off, 16)]
            r, last = _dup_rank(ci, tmp16_v)
            g = plsc.load_gather(offb_v, [ci])
            pos = g + r
            cat = base + off + iota
            posb_v[j, 16 * t:16 * (t + 1)] = pos
            catb_v[j, 16 * t:16 * (t + 1)] = cat
            plsc.store_scatter(offb_v, [ci], pos + 1, mask=last)
        return 0

    lax.fori_loop(0, SEG // 128, phase_c, 0)

    def scat_row(j, _):
        pltpu.sync_copy(catb_v.at[j], outsort_sh.at[posb_v.at[j]])
        return 0

    lax.fori_loop(0, SEG // 128, scat_row, 0)
    plsc.subcore_barrier()

    @pl.when(w < 4)
    def _():
        pltpu.sync_copy(outsort_sh.at[pl.ds(w * 4096, 4096)],
                        idx_hbm.at[pl.ds(w * 4096, 4096)])


def _make_k2():
    return functools.partial(
        pl.kernel,
        out_type=jax.ShapeDtypeStruct((KSEL,), jnp.int32),
        mesh=plsc.VectorSubcoreMesh(core_axis_name="c", subcore_axis_name="s",
                                    num_cores=1, num_subcores=16),
        compiler_params=pltpu.CompilerParams(needs_layout_passes=False),
        scratch_types=[
            pltpu.VMEM((SEG,), jnp.float32),
            pltpu.VMEM((SEG,), jnp.float32),
            pltpu.VMEM((SEG,), jnp.int32),
            pltpu.VMEM((NBINS,), jnp.int32),
            pltpu.VMEM((16 * NBINS,), jnp.int32),
            pltpu.VMEM((NBINS,), jnp.int32),
            pltpu.VMEM((NBINS,), jnp.int32),
            pltpu.VMEM((SEG // 128, 128), jnp.int32),
            pltpu.VMEM((SEG // 128, 128), jnp.int32),
            pltpu.VMEM((16,), jnp.int32),
            pltpu.VMEM_SHARED((16 * NBINS,), jnp.int32),
            pltpu.VMEM_SHARED((NPAD,), jnp.int32),
            pltpu.SemaphoreType.DMA,
        ],
    )(_k2_body)


def _k3_body(idx_hbm, nodes_hbm, out_hbm, idx_v, rows_v, sem):
    c = lax.axis_index("c")
    s = lax.axis_index("s")
    wid = c * 16 + s
    pltpu.sync_copy(idx_hbm.at[pl.ds(wid * 4, 4)], idx_v)
    for j in range(4):
        pltpu.async_copy(nodes_hbm.at[idx_v.at[j]],
                         rows_v.at[pl.ds(j * 128, 128)], sem).wait()
    pltpu.sync_copy(rows_v, out_hbm.at[pl.ds(wid * 512, 512)])


def _make_k3():
    return functools.partial(
        pl.kernel,
        out_type=jax.ShapeDtypeStruct((KSEL, D), jnp.float32),
        mesh=plsc.VectorSubcoreMesh(core_axis_name="c", subcore_axis_name="s",
                                    num_cores=2, num_subcores=16),
        compiler_params=pltpu.CompilerParams(needs_layout_passes=False),
        scratch_types=[
            pltpu.VMEM((4, 128), jnp.int32),
            pltpu.VMEM((512, D), jnp.float32),
            pltpu.SemaphoreType.DMA,
        ],
    )(_k3_body)


@functools.cache
def _kernels():
    return _make_k1(), _make_k2(), _make_k3()


def kernel(f, target_number_points, nodes):
    probs = f / jnp.sum(f)
    cdf = jnp.cumsum(probs)
    u = jax.random.uniform(jax.random.key(42), (TOTAL,),
                           dtype=jnp.float32) * cdf[-1]
    u_pad = jnp.concatenate(
        [u, jnp.full((UPAD - TOTAL,), jnp.inf, jnp.float32)])
    cdf_pad = jnp.concatenate([cdf, jnp.zeros((NPAD - N,), jnp.float32)])
    zeros_seg = jnp.zeros((SEG,), jnp.float32)
    ones128 = jnp.ones((128,), jnp.float32)
    k1, k2, k3 = _kernels()
    counts2 = k1(cdf_pad, u_pad, zeros_seg, ones128)
    sel = k2(counts2)
    sel = (sel + (target_number_points - KSEL)).astype(jnp.int32)
    return k3(sel.reshape(128, 128), nodes)
